# speedup vs baseline: 1.4201x; 1.4201x over previous
import jax
import jax.numpy as jnp
from jax.experimental import pallas as pl

DIM = 4096
N_EXPERTS = 64
TOP_K = 8
TOKEN_BLOCK = 512


def _gate_kernel(x_ref, wt_ref, bias_ref, w_out_ref, idx_out_ref):
    x = x_ref[...]
    wt = wt_ref[...]
    logits = jax.lax.dot_general(
        x, wt, (((1,), (0,)), ((), ())), preferred_element_type=jnp.float32
    )
    sp = jnp.maximum(logits, 0.0) + jnp.log1p(jnp.exp(-jnp.abs(logits)))
    scores = jnp.sqrt(sp)
    biased = scores + bias_ref[...]

    lane = jax.lax.broadcasted_iota(jnp.int32, biased.shape, 1)
    cur = biased
    neg_inf = jnp.float32(-jnp.inf)
    w_cols = []
    i_cols = []
    for _ in range(TOP_K):
        m = jnp.max(cur, axis=1, keepdims=True)
        is_max = cur == m
        idx = jnp.min(jnp.where(is_max, lane, N_EXPERTS), axis=1, keepdims=True)
        onehot = lane == idx
        w_cols.append(jnp.sum(jnp.where(onehot, scores, 0.0), axis=1, keepdims=True))
        i_cols.append(idx)
        cur = jnp.where(onehot, neg_inf, cur)

    w = jnp.concatenate(w_cols, axis=1)
    idx = jnp.concatenate(i_cols, axis=1)
    w = w / jnp.sum(w, axis=1, keepdims=True)
    w_out_ref[...] = w
    idx_out_ref[...] = idx


@jax.jit
def kernel(x, weight, bias):
    n_tokens = x.shape[0]
    wt = weight.T
    bias2 = bias.reshape(1, N_EXPERTS)
    grid = (n_tokens // TOKEN_BLOCK,)
    w, idx = pl.pallas_call(
        _gate_kernel,
        grid=grid,
        in_specs=[
            pl.BlockSpec((TOKEN_BLOCK, DIM), lambda i: (i, 0)),
            pl.BlockSpec((DIM, N_EXPERTS), lambda i: (0, 0)),
            pl.BlockSpec((1, N_EXPERTS), lambda i: (0, 0)),
        ],
        out_specs=[
            pl.BlockSpec((TOKEN_BLOCK, TOP_K), lambda i: (i, 0)),
            pl.BlockSpec((TOKEN_BLOCK, TOP_K), lambda i: (i, 0)),
        ],
        out_shape=[
            jax.ShapeDtypeStruct((n_tokens, TOP_K), jnp.float32),
            jax.ShapeDtypeStruct((n_tokens, TOP_K), jnp.int32),
        ],
    )(x, wt, bias2)
    return w, idx


# speedup vs baseline: 2.2066x; 1.5538x over previous
import jax
import jax.numpy as jnp
from jax.experimental import pallas as pl

DIM = 4096
N_EXPERTS = 64
TOP_K = 8
TOKEN_BLOCK = 512


def _gate_kernel(w_ref, x_ref, bias_ref, w_out_ref, idx_out_ref):
    w = w_ref[...]
    x = x_ref[...]
    logits = jax.lax.dot_general(
        w, x, (((1,), (1,)), ((), ())), preferred_element_type=jnp.float32
    )
    sp = jnp.maximum(logits, 0.0) + jnp.log1p(jnp.exp(-jnp.abs(logits)))
    scores = jnp.sqrt(sp)
    biased = scores + bias_ref[...]

    row = jax.lax.broadcasted_iota(jnp.int32, biased.shape, 0)
    rev_row_f = jnp.float32(N_EXPERTS - 1) - row.astype(jnp.float32)
    cur = biased
    neg_inf = jnp.float32(-jnp.inf)
    w_rows = []
    i_rows = []
    for _ in range(TOP_K):
        m = jnp.max(cur, axis=0, keepdims=True)
        is_max = cur == m
        rev = jnp.max(jnp.where(is_max, rev_row_f, -1.0), axis=0, keepdims=True)
        onehot = rev_row_f == rev
        w_rows.append(jnp.sum(jnp.where(onehot, scores, 0.0), axis=0, keepdims=True))
        i_rows.append(jnp.float32(N_EXPERTS - 1) - rev)
        cur = jnp.where(onehot, neg_inf, cur)

    wsel = jnp.concatenate(w_rows, axis=0)
    idx = jnp.concatenate(i_rows, axis=0).astype(jnp.int32)
    wsel = wsel / jnp.sum(wsel, axis=0, keepdims=True)
    w_out_ref[...] = wsel
    idx_out_ref[...] = idx


@jax.jit
def kernel(x, weight, bias):
    n_tokens = x.shape[0]
    bias2 = bias.reshape(N_EXPERTS, 1)
    grid = (n_tokens // TOKEN_BLOCK,)
    wsel, idx = pl.pallas_call(
        _gate_kernel,
        grid=grid,
        in_specs=[
            pl.BlockSpec((N_EXPERTS, DIM), lambda i: (0, 0)),
            pl.BlockSpec((TOKEN_BLOCK, DIM), lambda i: (i, 0)),
            pl.BlockSpec((N_EXPERTS, 1), lambda i: (0, 0)),
        ],
        out_specs=[
            pl.BlockSpec((TOP_K, TOKEN_BLOCK), lambda i: (0, i)),
            pl.BlockSpec((TOP_K, TOKEN_BLOCK), lambda i: (0, i)),
        ],
        out_shape=[
            jax.ShapeDtypeStruct((TOP_K, n_tokens), jnp.float32),
            jax.ShapeDtypeStruct((TOP_K, n_tokens), jnp.int32),
        ],
    )(weight, x, bias2)
    return wsel.T, idx.T


# speedup vs baseline: 2.3358x; 1.0586x over previous
import jax
import jax.numpy as jnp
from jax.experimental import pallas as pl

DIM = 4096
N_EXPERTS = 64
TOP_K = 8
TOKEN_BLOCK = 1024


def _gate_kernel(w_ref, x_ref, bias_ref, w_out_ref, idx_out_ref):
    w = w_ref[...]
    x = x_ref[...]
    logits = jax.lax.dot_general(
        w, x, (((1,), (1,)), ((), ())), preferred_element_type=jnp.float32
    )
    sp = jnp.maximum(logits, 0.0) + jnp.log1p(jnp.exp(-jnp.abs(logits)))
    scores = jnp.sqrt(sp)
    biased = scores + bias_ref[...]

    row = jax.lax.broadcasted_iota(jnp.int32, biased.shape, 0)
    rev_row_f = jnp.float32(N_EXPERTS - 1) - row.astype(jnp.float32)
    cur = biased
    neg_inf = jnp.float32(-jnp.inf)
    w_rows = []
    i_rows = []
    for _ in range(TOP_K):
        m = jnp.max(cur, axis=0, keepdims=True)
        is_max = cur == m
        rev = jnp.max(jnp.where(is_max, rev_row_f, -1.0), axis=0, keepdims=True)
        onehot = rev_row_f == rev
        w_rows.append(jnp.sum(jnp.where(onehot, scores, 0.0), axis=0, keepdims=True))
        i_rows.append(jnp.float32(N_EXPERTS - 1) - rev)
        cur = jnp.where(onehot, neg_inf, cur)

    wsel = jnp.concatenate(w_rows, axis=0)
    idx = jnp.concatenate(i_rows, axis=0).astype(jnp.int32)
    wsel = wsel / jnp.sum(wsel, axis=0, keepdims=True)
    w_out_ref[...] = wsel
    idx_out_ref[...] = idx


@jax.jit
def kernel(x, weight, bias):
    n_tokens = x.shape[0]
    bias2 = bias.reshape(N_EXPERTS, 1)
    grid = (n_tokens // TOKEN_BLOCK,)
    wsel, idx = pl.pallas_call(
        _gate_kernel,
        grid=grid,
        in_specs=[
            pl.BlockSpec((N_EXPERTS, DIM), lambda i: (0, 0)),
            pl.BlockSpec((TOKEN_BLOCK, DIM), lambda i: (i, 0)),
            pl.BlockSpec((N_EXPERTS, 1), lambda i: (0, 0)),
        ],
        out_specs=[
            pl.BlockSpec((TOP_K, TOKEN_BLOCK), lambda i: (0, i)),
            pl.BlockSpec((TOP_K, TOKEN_BLOCK), lambda i: (0, i)),
        ],
        out_shape=[
            jax.ShapeDtypeStruct((TOP_K, n_tokens), jnp.float32),
            jax.ShapeDtypeStruct((TOP_K, n_tokens), jnp.int32),
        ],
    )(weight, x, bias2)
    return wsel.T, idx.T
